# baseline (speedup 1.0000x reference)
import functools

import jax
import jax.numpy as jnp
from jax import lax
from jax.experimental import pallas as pl
from jax.experimental.pallas import tpu as pltpu
from jax.experimental.pallas import tpu_sc as plsc

N = 10000
E = 320000
D = 128
F1, F2, F3 = 128, 64, 32
T = 16
BN = 16

NP = 10112
N2 = 2 * NP
NC = 2
NS = 16
EPT = E // NS
K = 128
CH = 160
G = CH // 4
EPTP = CH * K
RPT = NP // NS
WD = 128
FW = 128

_mesh = plsc.VectorSubcoreMesh(core_axis_name="c", subcore_axis_name="s")


@functools.partial(
    pl.kernel,
    out_type=jax.ShapeDtypeStruct((NC, NP, FW), jnp.float32),
    mesh=_mesh,
    scratch_types=[
        pltpu.VMEM((4, 2, K), jnp.int32),
        pltpu.VMEM((4, 2, K), jnp.int32),
        pltpu.VMEM((K, FW), jnp.float32),
        pltpu.VMEM((K, FW), jnp.float32),
        pltpu.VMEM_SHARED((NP, FW), jnp.float32),
        pltpu.SemaphoreType.DMA,
        pltpu.SemaphoreType.DMA,
        pltpu.SemaphoreType.DMA,
        pltpu.SemaphoreType.DMA,
        pltpu.SemaphoreType.DMA,
    ],
    name="edge_scatter",
)
def _edge_scatter(table, idx, zeros, out, ia, ib, r0, r1, acc,
                  isem, gs0, gs1, ss0, ss1):
    c = lax.axis_index("c")
    s = lax.axis_index("s")
    pltpu.sync_copy(zeros.at[pl.ds(s * RPT, RPT)], acc.at[pl.ds(s * RPT, RPT)])
    plsc.subcore_barrier()
    pltpu.async_copy(idx.at[c, s, 0], ia, isem)

    def process(x):
        g0 = pltpu.async_copy(table.at[x.at[0, 0]], r0, gs0)
        g1 = pltpu.async_copy(table.at[x.at[1, 0]], r1, gs1)
        g0.wait()
        s0 = pltpu.async_copy(r0, acc.at[x.at[0, 1]], ss0, add=True)
        g1.wait()
        s1 = pltpu.async_copy(r1, acc.at[x.at[1, 1]], ss1, add=True)
        s0.wait()
        g2 = pltpu.async_copy(table.at[x.at[2, 0]], r0, gs0)
        s1.wait()
        g3 = pltpu.async_copy(table.at[x.at[3, 0]], r1, gs1)
        g2.wait()
        s2 = pltpu.async_copy(r0, acc.at[x.at[2, 1]], ss0, add=True)
        g3.wait()
        s3 = pltpu.async_copy(r1, acc.at[x.at[3, 1]], ss1, add=True)
        s2.wait()
        s3.wait()

    def body(q, carry):
        ga = 2 * q
        pltpu.make_async_copy(idx.at[c, s, ga], ia, isem).wait()
        pltpu.async_copy(idx.at[c, s, ga + 1], ib, isem)
        process(ia)
        pltpu.make_async_copy(idx.at[c, s, ga + 1], ib, isem).wait()
        pltpu.async_copy(idx.at[c, s, ga + 2], ia, isem)
        process(ib)
        return carry

    lax.fori_loop(0, G // 2, body, 0)
    pltpu.make_async_copy(idx.at[c, s, G], ia, isem).wait()
    plsc.subcore_barrier()
    pltpu.sync_copy(acc.at[pl.ds(s * RPT, RPT)], out.at[c, pl.ds(s * RPT, RPT)])


@functools.partial(
    pl.kernel,
    out_type=jax.ShapeDtypeStruct((NC, NP, WD), jnp.float32),
    mesh=_mesh,
    scratch_types=[
        pltpu.VMEM((CH, K), jnp.int32),
        pltpu.VMEM((K, WD), jnp.float32),
        pltpu.VMEM_SHARED((NP, WD), jnp.float32),
        pltpu.SemaphoreType.DMA,
        pltpu.SemaphoreType.DMA,
    ],
    name="deg_scatter",
)
def _deg_scatter(dst, ones, zeros, out, dst_v, ones_v, acc, ss0, ss1):
    c = lax.axis_index("c")
    s = lax.axis_index("s")
    pltpu.sync_copy(zeros.at[pl.ds(s * RPT, RPT)], acc.at[pl.ds(s * RPT, RPT)])
    pltpu.sync_copy(dst.at[c, s], dst_v)
    pltpu.sync_copy(ones, ones_v)
    plsc.subcore_barrier()

    def body(q, carry):
        s0 = pltpu.async_copy(ones_v, acc.at[dst_v.at[2 * q]], ss0, add=True)
        s1 = pltpu.async_copy(ones_v, acc.at[dst_v.at[2 * q + 1]], ss1, add=True)
        s0.wait()
        s1.wait()
        return carry

    lax.fori_loop(0, CH // 2, body, 0)
    plsc.subcore_barrier()
    pltpu.sync_copy(acc.at[pl.ds(s * RPT, RPT)], out.at[c, pl.ds(s * RPT, RPT)])


def _first_body(deg_ref, x_ref, w_ref, dinv_ref, g_ref):
    dinv = lax.rsqrt(deg_ref[:, 0:1] + 1.0)
    dinv_ref[...] = dinv
    g_ref[...] = dinv * jnp.dot(x_ref[...], w_ref[...],
                                preferred_element_type=jnp.float32)


def _make_mid_body(fin, fout):
    def mid(agg_ref, g_ref, dinv_ref, b_ref, w_ref, gn_ref):
        dinv = dinv_ref[...]
        h = jnp.maximum(
            dinv * (agg_ref[:, :fin] + g_ref[:, :fin]) + b_ref[...], 0.0)
        gn = dinv * jnp.dot(h, w_ref[...], preferred_element_type=jnp.float32)
        gn_ref[...] = jnp.concatenate(
            [gn, jnp.zeros((N2, FW - fout), jnp.float32)], axis=1)
    return mid


def _final_body(agg_ref, g_ref, dinv_ref, b_ref, f_ref):
    f_ref[...] = (dinv_ref[...] * (agg_ref[:, :F3] + g_ref[:, :F3])
                  + b_ref[...])


def _tc(body, out_shape, *args):
    return pl.pallas_call(body, out_shape=out_shape)(*args)


def _prep_edges(ei, off):
    src = ei[0].reshape(NS, EPT)
    dst = ei[1].reshape(NS, EPT)
    src = jnp.pad(src, ((0, 0), (0, EPTP - EPT))) + off
    dst = jnp.pad(dst, ((0, 0), (0, EPTP - EPT)), constant_values=N)
    idx = jnp.stack([src.reshape(NS, G, 4, K), dst.reshape(NS, G, 4, K)],
                    axis=3)
    idx = jnp.pad(idx, ((0, 0), (0, 1), (0, 0), (0, 0), (0, 0)))
    return idx, dst.reshape(NS, CH, K)


def kernel(features_1, edge_index_1, features_2, edge_index_2, W1, b1, W2, b2,
           W3, b3, att_W, ntn_W, ntn_V, ntn_b, fc_W, fc_b, sc_W, sc_b):
    f32 = jnp.float32
    zrows = jnp.zeros((NP - N, D), f32)
    x_both = jnp.concatenate([features_1, zrows, features_2, zrows], axis=0)

    i1, d1 = _prep_edges(edge_index_1, 0)
    i2, d2 = _prep_edges(edge_index_2, NP)
    idx = jnp.stack([i1, i2])
    dst = jnp.stack([d1, d2])
    zt = jnp.zeros((NP, FW), f32)

    deg = _deg_scatter(dst, jnp.ones((K, WD), f32), jnp.zeros((NP, WD), f32))
    deg = deg.reshape(N2, WD)

    dinv, g1 = _tc(
        _first_body,
        (jax.ShapeDtypeStruct((N2, 1), f32), jax.ShapeDtypeStruct((N2, F1), f32)),
        deg, x_both, W1)

    a1 = _edge_scatter(g1, idx, zt)
    g2 = _tc(_make_mid_body(F1, F2), jax.ShapeDtypeStruct((N2, FW), f32),
             a1.reshape(N2, FW), g1, dinv, b1[None, :], W2)

    a2 = _edge_scatter(g2, idx, zt)
    g3 = _tc(_make_mid_body(F2, F3), jax.ShapeDtypeStruct((N2, FW), f32),
             a2.reshape(N2, FW), g2, dinv, b2[None, :], W3)

    a3 = _edge_scatter(g3, idx, zt)
    f = _tc(_final_body, jax.ShapeDtypeStruct((N2, F3), f32),
            a3.reshape(N2, FW), g3, dinv, b3[None, :])

    def _pool(h):
        gc = jnp.tanh(jnp.mean(h @ att_W, axis=0))
        sg = jax.nn.sigmoid(h @ gc[:, None])
        return h.T @ sg

    p1 = _pool(f[0:N])
    p2 = _pool(f[NP:NP + N])
    scoring = (p1.T @ ntn_W.reshape(F3, -1)).reshape(F3, -1)
    scoring = scoring.T @ p2
    comb = jnp.concatenate([p1, p2], axis=0)
    block = ntn_V @ comb
    s = jax.nn.relu(scoring + block + ntn_b).T
    s = jax.nn.relu(s @ fc_W + fc_b)
    return jax.nn.sigmoid(s @ sc_W + sc_b)


# speedup vs baseline: 1.1980x; 1.1980x over previous
import functools

import jax
import jax.numpy as jnp
from jax import lax
from jax.experimental import pallas as pl
from jax.experimental.pallas import tpu as pltpu
from jax.experimental.pallas import tpu_sc as plsc

N = 10000
E = 320000
D = 128
F1, F2, F3 = 128, 64, 32
T = 16
BN = 16

NP = 10112
N2 = 2 * NP
NC = 2
NS = 16
EPT = E // NS
K = 128
CH = (EPT + K - 1) // K
EPTP = CH * K
RPT = NP // NS
WD = 128
FW = 128

_mesh = plsc.VectorSubcoreMesh(core_axis_name="c", subcore_axis_name="s")


@functools.partial(
    pl.kernel,
    out_type=jax.ShapeDtypeStruct((NC, NP, FW), jnp.float32),
    mesh=_mesh,
    scratch_types=[
        pltpu.VMEM((K,), jnp.int32),
        pltpu.VMEM((CH, K), jnp.int32),
        pltpu.VMEM((K, FW), jnp.float32),
        pltpu.VMEM_SHARED((NP, FW), jnp.float32),
        pltpu.SemaphoreType.DMA,
    ],
    name="edge_scatter",
)
def _edge_scatter(table, src, dst, zeros, out, src_v, dst_v, rows_v, acc, sem):
    c = lax.axis_index("c")
    s = lax.axis_index("s")
    pltpu.sync_copy(zeros.at[pl.ds(s * RPT, RPT)], acc.at[pl.ds(s * RPT, RPT)])
    pltpu.sync_copy(dst.at[c, s], dst_v)
    plsc.subcore_barrier()

    def body(j, carry):
        pltpu.sync_copy(src.at[c, s, j], src_v)
        pltpu.async_copy(table.at[src_v], rows_v, sem).wait()
        pltpu.sync_copy(rows_v, acc.at[dst_v.at[j]], add=True)
        return carry

    lax.fori_loop(0, CH, body, 0)
    plsc.subcore_barrier()
    pltpu.sync_copy(acc.at[pl.ds(s * RPT, RPT)], out.at[c, pl.ds(s * RPT, RPT)])


@functools.partial(
    pl.kernel,
    out_type=jax.ShapeDtypeStruct((NC, NP, WD), jnp.float32),
    mesh=_mesh,
    scratch_types=[
        pltpu.VMEM((CH, K), jnp.int32),
        pltpu.VMEM((K, WD), jnp.float32),
        pltpu.VMEM_SHARED((NP, WD), jnp.float32),
        pltpu.SemaphoreType.DMA,
        pltpu.SemaphoreType.DMA,
    ],
    name="deg_scatter",
)
def _deg_scatter(dst, ones, zeros, out, dst_v, ones_v, acc, ss0, ss1):
    c = lax.axis_index("c")
    s = lax.axis_index("s")
    pltpu.sync_copy(zeros.at[pl.ds(s * RPT, RPT)], acc.at[pl.ds(s * RPT, RPT)])
    pltpu.sync_copy(dst.at[c, s], dst_v)
    pltpu.sync_copy(ones, ones_v)
    plsc.subcore_barrier()

    def body(q, carry):
        s0 = pltpu.async_copy(ones_v, acc.at[dst_v.at[2 * q]], ss0, add=True)
        s1 = pltpu.async_copy(ones_v, acc.at[dst_v.at[2 * q + 1]], ss1, add=True)
        s0.wait()
        s1.wait()
        return carry

    lax.fori_loop(0, CH // 2, body, 0)
    plsc.subcore_barrier()
    pltpu.sync_copy(acc.at[pl.ds(s * RPT, RPT)], out.at[c, pl.ds(s * RPT, RPT)])


def _first_body(deg_ref, x_ref, w_ref, dinv_ref, g_ref):
    dinv = lax.rsqrt(deg_ref[:, 0:1] + 1.0)
    dinv_ref[...] = dinv
    g_ref[...] = dinv * jnp.dot(x_ref[...], w_ref[...],
                                preferred_element_type=jnp.float32)


def _make_mid_body(fin, fout):
    def mid(agg_ref, g_ref, dinv_ref, b_ref, w_ref, gn_ref):
        dinv = dinv_ref[...]
        h = jnp.maximum(
            dinv * (agg_ref[:, :fin] + g_ref[:, :fin]) + b_ref[...], 0.0)
        gn = dinv * jnp.dot(h, w_ref[...], preferred_element_type=jnp.float32)
        if fout < FW:
            gn = jnp.concatenate(
                [gn, jnp.zeros((N2, FW - fout), jnp.float32)], axis=1)
        gn_ref[...] = gn
    return mid


def _final_body(agg_ref, g_ref, dinv_ref, b_ref, f_ref):
    f_ref[...] = (dinv_ref[...] * (agg_ref[:, :F3] + g_ref[:, :F3])
                  + b_ref[...])


def _tc(body, out_shape, *args):
    return pl.pallas_call(body, out_shape=out_shape)(*args)


def _prep_edges(ei):
    src = ei[0].reshape(NS, EPT)
    dst = ei[1].reshape(NS, EPT)
    src = jnp.pad(src, ((0, 0), (0, EPTP - EPT)))
    dst = jnp.pad(dst, ((0, 0), (0, EPTP - EPT)), constant_values=N)
    return src.reshape(NS, CH, K), dst.reshape(NS, CH, K)


def kernel(features_1, edge_index_1, features_2, edge_index_2, W1, b1, W2, b2,
           W3, b3, att_W, ntn_W, ntn_V, ntn_b, fc_W, fc_b, sc_W, sc_b):
    f32 = jnp.float32
    zrows = jnp.zeros((NP - N, D), f32)
    x_both = jnp.concatenate([features_1, zrows, features_2, zrows], axis=0)

    s1, d1 = _prep_edges(edge_index_1)
    s2, d2 = _prep_edges(edge_index_2)
    srcg = jnp.stack([s1, s2 + NP])
    dst = jnp.stack([d1, d2])
    zt = jnp.zeros((NP, FW), f32)

    deg = _deg_scatter(dst, jnp.ones((K, WD), f32), jnp.zeros((NP, WD), f32))
    deg = deg.reshape(N2, WD)

    dinv, g1 = _tc(
        _first_body,
        (jax.ShapeDtypeStruct((N2, 1), f32), jax.ShapeDtypeStruct((N2, F1), f32)),
        deg, x_both, W1)

    a1 = _edge_scatter(g1, srcg, dst, zt)
    g2 = _tc(_make_mid_body(F1, F2), jax.ShapeDtypeStruct((N2, FW), f32),
             a1.reshape(N2, FW), g1, dinv, b1[None, :], W2)

    a2 = _edge_scatter(g2, srcg, dst, zt)
    g3 = _tc(_make_mid_body(F2, F3), jax.ShapeDtypeStruct((N2, FW), f32),
             a2.reshape(N2, FW), g2, dinv, b2[None, :], W3)

    a3 = _edge_scatter(g3, srcg, dst, zt)
    f = _tc(_final_body, jax.ShapeDtypeStruct((N2, F3), f32),
            a3.reshape(N2, FW), g3, dinv, b3[None, :])

    def _pool(h):
        gc = jnp.tanh(jnp.mean(h @ att_W, axis=0))
        sg = jax.nn.sigmoid(h @ gc[:, None])
        return h.T @ sg

    p1 = _pool(f[0:N])
    p2 = _pool(f[NP:NP + N])
    scoring = (p1.T @ ntn_W.reshape(F3, -1)).reshape(F3, -1)
    scoring = scoring.T @ p2
    comb = jnp.concatenate([p1, p2], axis=0)
    block = ntn_V @ comb
    s = jax.nn.relu(scoring + block + ntn_b).T
    s = jax.nn.relu(s @ fc_W + fc_b)
    return jax.nn.sigmoid(s @ sc_W + sc_b)


# speedup vs baseline: 1.2128x; 1.0123x over previous
import functools

import jax
import jax.numpy as jnp
from jax import lax
from jax.experimental import pallas as pl
from jax.experimental.pallas import tpu as pltpu
from jax.experimental.pallas import tpu_sc as plsc

N = 10000
E = 320000
D = 128
F1, F2, F3 = 128, 64, 32
T = 16
BN = 16

NP = 10112
N2 = 2 * NP
NC = 2
NS = 16
EPT = E // NS
K = 128
CH = 158
CH2 = CH // 2
EPTP = CH * K
RPT = NP // NS
WD = 128
FW = 128

_mesh = plsc.VectorSubcoreMesh(core_axis_name="c", subcore_axis_name="s")


@functools.partial(
    pl.kernel,
    out_type=jax.ShapeDtypeStruct((NC, NP, FW), jnp.float32),
    mesh=_mesh,
    scratch_types=[
        pltpu.VMEM((2, 2, K), jnp.int32),
        pltpu.VMEM((K, FW), jnp.float32),
        pltpu.VMEM((K, FW), jnp.float32),
        pltpu.VMEM_SHARED((NP, FW), jnp.float32),
        pltpu.SemaphoreType.DMA,
        pltpu.SemaphoreType.DMA,
        pltpu.SemaphoreType.DMA,
    ],
    name="edge_scatter",
)
def _edge_scatter(table, idx, zeros, out, iv, r0, r1, acc, gs, ss0, ss1):
    c = lax.axis_index("c")
    s = lax.axis_index("s")
    pltpu.sync_copy(zeros.at[pl.ds(s * RPT, RPT)], acc.at[pl.ds(s * RPT, RPT)])
    plsc.subcore_barrier()

    pltpu.sync_copy(idx.at[c, s, 0], iv)
    pltpu.async_copy(table.at[iv.at[0, 0]], r0, gs).wait()
    pltpu.async_copy(r0, acc.at[iv.at[0, 1]], ss0, add=True)
    pltpu.async_copy(table.at[iv.at[1, 0]], r1, gs).wait()
    pltpu.async_copy(r1, acc.at[iv.at[1, 1]], ss1, add=True)

    def body(q, carry):
        pltpu.make_async_copy(r0, acc.at[iv.at[0, 1]], ss0).wait()
        pltpu.make_async_copy(r1, acc.at[iv.at[1, 1]], ss1).wait()
        pltpu.sync_copy(idx.at[c, s, q], iv)
        pltpu.async_copy(table.at[iv.at[0, 0]], r0, gs).wait()
        pltpu.async_copy(r0, acc.at[iv.at[0, 1]], ss0, add=True)
        pltpu.async_copy(table.at[iv.at[1, 0]], r1, gs).wait()
        pltpu.async_copy(r1, acc.at[iv.at[1, 1]], ss1, add=True)
        return carry

    lax.fori_loop(1, CH2, body, 0)
    pltpu.make_async_copy(r0, acc.at[iv.at[0, 1]], ss0).wait()
    pltpu.make_async_copy(r1, acc.at[iv.at[1, 1]], ss1).wait()
    plsc.subcore_barrier()
    pltpu.sync_copy(acc.at[pl.ds(s * RPT, RPT)], out.at[c, pl.ds(s * RPT, RPT)])


@functools.partial(
    pl.kernel,
    out_type=jax.ShapeDtypeStruct((NC, NP, WD), jnp.float32),
    mesh=_mesh,
    scratch_types=[
        pltpu.VMEM((CH, K), jnp.int32),
        pltpu.VMEM((K, WD), jnp.float32),
        pltpu.VMEM_SHARED((NP, WD), jnp.float32),
        pltpu.SemaphoreType.DMA,
        pltpu.SemaphoreType.DMA,
    ],
    name="deg_scatter",
)
def _deg_scatter(dst, ones, zeros, out, dst_v, ones_v, acc, ss0, ss1):
    c = lax.axis_index("c")
    s = lax.axis_index("s")
    pltpu.sync_copy(zeros.at[pl.ds(s * RPT, RPT)], acc.at[pl.ds(s * RPT, RPT)])
    pltpu.sync_copy(dst.at[c, s], dst_v)
    pltpu.sync_copy(ones, ones_v)
    plsc.subcore_barrier()

    pltpu.async_copy(ones_v, acc.at[dst_v.at[0]], ss0, add=True)
    pltpu.async_copy(ones_v, acc.at[dst_v.at[1]], ss1, add=True)

    def body(q, carry):
        pltpu.make_async_copy(ones_v, acc.at[dst_v.at[0]], ss0).wait()
        pltpu.async_copy(ones_v, acc.at[dst_v.at[2 * q]], ss0, add=True)
        pltpu.make_async_copy(ones_v, acc.at[dst_v.at[1]], ss1).wait()
        pltpu.async_copy(ones_v, acc.at[dst_v.at[2 * q + 1]], ss1, add=True)
        return carry

    lax.fori_loop(1, CH2, body, 0)
    pltpu.make_async_copy(ones_v, acc.at[dst_v.at[0]], ss0).wait()
    pltpu.make_async_copy(ones_v, acc.at[dst_v.at[1]], ss1).wait()
    plsc.subcore_barrier()
    pltpu.sync_copy(acc.at[pl.ds(s * RPT, RPT)], out.at[c, pl.ds(s * RPT, RPT)])


def _first_body(deg_ref, x_ref, w_ref, dinv_ref, g_ref):
    dinv = lax.rsqrt(deg_ref[:, 0:1] + 1.0)
    dinv_ref[...] = dinv
    g_ref[...] = dinv * jnp.dot(x_ref[...], w_ref[...],
                                preferred_element_type=jnp.float32)


def _make_mid_body(fin, fout):
    def mid(agg_ref, g_ref, dinv_ref, b_ref, w_ref, gn_ref):
        dinv = dinv_ref[...]
        h = jnp.maximum(
            dinv * (agg_ref[:, :fin] + g_ref[:, :fin]) + b_ref[...], 0.0)
        gn = dinv * jnp.dot(h, w_ref[...], preferred_element_type=jnp.float32)
        if fout < FW:
            gn = jnp.concatenate(
                [gn, jnp.zeros((N2, FW - fout), jnp.float32)], axis=1)
        gn_ref[...] = gn
    return mid


def _final_body(agg_ref, g_ref, dinv_ref, b_ref, f_ref):
    f_ref[...] = (dinv_ref[...] * (agg_ref[:, :F3] + g_ref[:, :F3])
                  + b_ref[...])


def _tc(body, out_shape, *args):
    return pl.pallas_call(body, out_shape=out_shape)(*args)


def _prep_edges(ei, off):
    src = ei[0].reshape(NS, EPT)
    dst = ei[1].reshape(NS, EPT)
    src = jnp.pad(src, ((0, 0), (0, EPTP - EPT))) + off
    dst = jnp.pad(dst, ((0, 0), (0, EPTP - EPT)), constant_values=N)
    idx = jnp.stack([src.reshape(NS, CH2, 2, K), dst.reshape(NS, CH2, 2, K)],
                    axis=3)
    return idx, dst.reshape(NS, CH, K)


def kernel(features_1, edge_index_1, features_2, edge_index_2, W1, b1, W2, b2,
           W3, b3, att_W, ntn_W, ntn_V, ntn_b, fc_W, fc_b, sc_W, sc_b):
    f32 = jnp.float32
    zrows = jnp.zeros((NP - N, D), f32)
    x_both = jnp.concatenate([features_1, zrows, features_2, zrows], axis=0)

    i1, d1 = _prep_edges(edge_index_1, 0)
    i2, d2 = _prep_edges(edge_index_2, NP)
    idx = jnp.stack([i1, i2])
    dst = jnp.stack([d1, d2])
    zt = jnp.zeros((NP, FW), f32)

    deg = _deg_scatter(dst, jnp.ones((K, WD), f32), jnp.zeros((NP, WD), f32))
    deg = deg.reshape(N2, WD)

    dinv, g1 = _tc(
        _first_body,
        (jax.ShapeDtypeStruct((N2, 1), f32), jax.ShapeDtypeStruct((N2, F1), f32)),
        deg, x_both, W1)

    a1 = _edge_scatter(g1, idx, zt)
    g2 = _tc(_make_mid_body(F1, F2), jax.ShapeDtypeStruct((N2, FW), f32),
             a1.reshape(N2, FW), g1, dinv, b1[None, :], W2)

    a2 = _edge_scatter(g2, idx, zt)
    g3 = _tc(_make_mid_body(F2, F3), jax.ShapeDtypeStruct((N2, FW), f32),
             a2.reshape(N2, FW), g2, dinv, b2[None, :], W3)

    a3 = _edge_scatter(g3, idx, zt)
    f = _tc(_final_body, jax.ShapeDtypeStruct((N2, F3), f32),
            a3.reshape(N2, FW), g3, dinv, b3[None, :])

    def _pool(h):
        gc = jnp.tanh(jnp.mean(h @ att_W, axis=0))
        sg = jax.nn.sigmoid(h @ gc[:, None])
        return h.T @ sg

    p1 = _pool(f[0:N])
    p2 = _pool(f[NP:NP + N])
    scoring = (p1.T @ ntn_W.reshape(F3, -1)).reshape(F3, -1)
    scoring = scoring.T @ p2
    comb = jnp.concatenate([p1, p2], axis=0)
    block = ntn_V @ comb
    s = jax.nn.relu(scoring + block + ntn_b).T
    s = jax.nn.relu(s @ fc_W + fc_b)
    return jax.nn.sigmoid(s @ sc_W + sc_b)


# speedup vs baseline: 1.2197x; 1.0057x over previous
import functools

import jax
import jax.numpy as jnp
from jax import lax
from jax.experimental import pallas as pl
from jax.experimental.pallas import tpu as pltpu
from jax.experimental.pallas import tpu_sc as plsc

N = 10000
E = 320000
D = 128
F1, F2, F3 = 128, 64, 32
T = 16
BN = 16

NP = 10112
N2 = 2 * NP
NC = 2
NS = 16
EPT = E // NS
K = 128
CH = 158
CH2 = CH // 2
EPTP = CH * K
RPT = NP // NS
WD = 128
FW = 128

_mesh = plsc.VectorSubcoreMesh(core_axis_name="c", subcore_axis_name="s")


@functools.partial(
    pl.kernel,
    out_type=jax.ShapeDtypeStruct((NC, NP, FW), jnp.float32),
    mesh=_mesh,
    scratch_types=[
        pltpu.VMEM((2, 2, K), jnp.int32),
        pltpu.VMEM((K, FW), jnp.float32),
        pltpu.VMEM((K, FW), jnp.float32),
        pltpu.VMEM_SHARED((NP, FW), jnp.float32),
        pltpu.SemaphoreType.DMA,
        pltpu.SemaphoreType.DMA,
        pltpu.SemaphoreType.DMA,
        pltpu.SemaphoreType.DMA,
    ],
    name="edge_scatter",
)
def _edge_scatter(table, idx, zeros, out, iv, r0, r1, acc, gs0, gs1, ss0, ss1):
    c = lax.axis_index("c")
    s = lax.axis_index("s")
    pltpu.sync_copy(zeros.at[pl.ds(s * RPT, RPT)], acc.at[pl.ds(s * RPT, RPT)])
    plsc.subcore_barrier()

    pltpu.sync_copy(idx.at[c, s, 0], iv)
    g0 = pltpu.async_copy(table.at[iv.at[0, 0]], r0, gs0)
    g1 = pltpu.async_copy(table.at[iv.at[1, 0]], r1, gs1)
    g0.wait()
    pltpu.async_copy(r0, acc.at[iv.at[0, 1]], ss0, add=True)
    g1.wait()
    pltpu.async_copy(r1, acc.at[iv.at[1, 1]], ss1, add=True)

    def body(q, carry):
        pltpu.make_async_copy(r0, acc.at[iv.at[0, 1]], ss0).wait()
        pltpu.make_async_copy(r1, acc.at[iv.at[1, 1]], ss1).wait()
        pltpu.sync_copy(idx.at[c, s, q], iv)
        g0 = pltpu.async_copy(table.at[iv.at[0, 0]], r0, gs0)
        g1 = pltpu.async_copy(table.at[iv.at[1, 0]], r1, gs1)
        g0.wait()
        pltpu.async_copy(r0, acc.at[iv.at[0, 1]], ss0, add=True)
        g1.wait()
        pltpu.async_copy(r1, acc.at[iv.at[1, 1]], ss1, add=True)
        return carry

    lax.fori_loop(1, CH2, body, 0)
    pltpu.make_async_copy(r0, acc.at[iv.at[0, 1]], ss0).wait()
    pltpu.make_async_copy(r1, acc.at[iv.at[1, 1]], ss1).wait()
    plsc.subcore_barrier()
    pltpu.sync_copy(acc.at[pl.ds(s * RPT, RPT)], out.at[c, pl.ds(s * RPT, RPT)])


@functools.partial(
    pl.kernel,
    out_type=jax.ShapeDtypeStruct((NC, NP, WD), jnp.float32),
    mesh=_mesh,
    scratch_types=[
        pltpu.VMEM((CH, K), jnp.int32),
        pltpu.VMEM((K, WD), jnp.float32),
        pltpu.VMEM_SHARED((NP, WD), jnp.float32),
        pltpu.SemaphoreType.DMA,
        pltpu.SemaphoreType.DMA,
    ],
    name="deg_scatter",
)
def _deg_scatter(dst, ones, zeros, out, dst_v, ones_v, acc, ss0, ss1):
    c = lax.axis_index("c")
    s = lax.axis_index("s")
    pltpu.sync_copy(zeros.at[pl.ds(s * RPT, RPT)], acc.at[pl.ds(s * RPT, RPT)])
    pltpu.sync_copy(dst.at[c, s], dst_v)
    pltpu.sync_copy(ones, ones_v)
    plsc.subcore_barrier()

    pltpu.async_copy(ones_v, acc.at[dst_v.at[0]], ss0, add=True)
    pltpu.async_copy(ones_v, acc.at[dst_v.at[1]], ss1, add=True)

    def body(q, carry):
        pltpu.make_async_copy(ones_v, acc.at[dst_v.at[0]], ss0).wait()
        pltpu.async_copy(ones_v, acc.at[dst_v.at[2 * q]], ss0, add=True)
        pltpu.make_async_copy(ones_v, acc.at[dst_v.at[1]], ss1).wait()
        pltpu.async_copy(ones_v, acc.at[dst_v.at[2 * q + 1]], ss1, add=True)
        return carry

    lax.fori_loop(1, CH2, body, 0)
    pltpu.make_async_copy(ones_v, acc.at[dst_v.at[0]], ss0).wait()
    pltpu.make_async_copy(ones_v, acc.at[dst_v.at[1]], ss1).wait()
    plsc.subcore_barrier()
    pltpu.sync_copy(acc.at[pl.ds(s * RPT, RPT)], out.at[c, pl.ds(s * RPT, RPT)])


def _first_body(deg_ref, x_ref, w_ref, dinv_ref, g_ref):
    dinv = lax.rsqrt(deg_ref[:, 0:1] + 1.0)
    dinv_ref[...] = dinv
    g_ref[...] = dinv * jnp.dot(x_ref[...], w_ref[...],
                                preferred_element_type=jnp.float32)


def _make_mid_body(fin, fout):
    def mid(agg_ref, g_ref, dinv_ref, b_ref, w_ref, gn_ref):
        dinv = dinv_ref[...]
        h = jnp.maximum(
            dinv * (agg_ref[:, :fin] + g_ref[:, :fin]) + b_ref[...], 0.0)
        gn = dinv * jnp.dot(h, w_ref[...], preferred_element_type=jnp.float32)
        if fout < FW:
            gn = jnp.concatenate(
                [gn, jnp.zeros((N2, FW - fout), jnp.float32)], axis=1)
        gn_ref[...] = gn
    return mid


def _final_body(agg_ref, g_ref, dinv_ref, b_ref, f_ref):
    f_ref[...] = (dinv_ref[...] * (agg_ref[:, :F3] + g_ref[:, :F3])
                  + b_ref[...])


def _tc(body, out_shape, *args):
    return pl.pallas_call(body, out_shape=out_shape)(*args)


def _prep_edges(ei, off):
    src = ei[0].reshape(NS, EPT)
    dst = ei[1].reshape(NS, EPT)
    src = jnp.pad(src, ((0, 0), (0, EPTP - EPT))) + off
    dst = jnp.pad(dst, ((0, 0), (0, EPTP - EPT)), constant_values=N)
    idx = jnp.stack([src.reshape(NS, CH2, 2, K), dst.reshape(NS, CH2, 2, K)],
                    axis=3)
    return idx, dst.reshape(NS, CH, K)


def kernel(features_1, edge_index_1, features_2, edge_index_2, W1, b1, W2, b2,
           W3, b3, att_W, ntn_W, ntn_V, ntn_b, fc_W, fc_b, sc_W, sc_b):
    f32 = jnp.float32
    zrows = jnp.zeros((NP - N, D), f32)
    x_both = jnp.concatenate([features_1, zrows, features_2, zrows], axis=0)

    i1, d1 = _prep_edges(edge_index_1, 0)
    i2, d2 = _prep_edges(edge_index_2, NP)
    idx = jnp.stack([i1, i2])
    dst = jnp.stack([d1, d2])
    zt = jnp.zeros((NP, FW), f32)

    deg = _deg_scatter(dst, jnp.ones((K, WD), f32), jnp.zeros((NP, WD), f32))
    deg = deg.reshape(N2, WD)

    dinv, g1 = _tc(
        _first_body,
        (jax.ShapeDtypeStruct((N2, 1), f32), jax.ShapeDtypeStruct((N2, F1), f32)),
        deg, x_both, W1)

    a1 = _edge_scatter(g1, idx, zt)
    g2 = _tc(_make_mid_body(F1, F2), jax.ShapeDtypeStruct((N2, FW), f32),
             a1.reshape(N2, FW), g1, dinv, b1[None, :], W2)

    a2 = _edge_scatter(g2, idx, zt)
    g3 = _tc(_make_mid_body(F2, F3), jax.ShapeDtypeStruct((N2, FW), f32),
             a2.reshape(N2, FW), g2, dinv, b2[None, :], W3)

    a3 = _edge_scatter(g3, idx, zt)
    f = _tc(_final_body, jax.ShapeDtypeStruct((N2, F3), f32),
            a3.reshape(N2, FW), g3, dinv, b3[None, :])

    def _pool(h):
        gc = jnp.tanh(jnp.mean(h @ att_W, axis=0))
        sg = jax.nn.sigmoid(h @ gc[:, None])
        return h.T @ sg

    p1 = _pool(f[0:N])
    p2 = _pool(f[NP:NP + N])
    scoring = (p1.T @ ntn_W.reshape(F3, -1)).reshape(F3, -1)
    scoring = scoring.T @ p2
    comb = jnp.concatenate([p1, p2], axis=0)
    block = ntn_V @ comb
    s = jax.nn.relu(scoring + block + ntn_b).T
    s = jax.nn.relu(s @ fc_W + fc_b)
    return jax.nn.sigmoid(s @ sc_W + sc_b)


# speedup vs baseline: 1.2220x; 1.0018x over previous
import functools

import jax
import jax.numpy as jnp
from jax import lax
from jax.experimental import pallas as pl
from jax.experimental.pallas import tpu as pltpu
from jax.experimental.pallas import tpu_sc as plsc

N = 10000
E = 320000
D = 128
F1, F2, F3 = 128, 64, 32
T = 16
BN = 16

NP = 10112
N2 = 2 * NP
NC = 2
NS = 16
EPT = E // NS
K = 128
CH = 158
CH2 = CH // 2
EPTP = CH * K
RPT = NP // NS
WD = 128
FW = 128

_mesh = plsc.VectorSubcoreMesh(core_axis_name="c", subcore_axis_name="s")


@functools.partial(
    pl.kernel,
    out_type=jax.ShapeDtypeStruct((NC, NP, FW), jnp.float32),
    mesh=_mesh,
    scratch_types=[
        pltpu.VMEM((2, 2, K), jnp.int32),
        pltpu.VMEM((K, FW), jnp.float32),
        pltpu.VMEM((K, FW), jnp.float32),
        pltpu.VMEM_SHARED((NP, FW), jnp.float32),
        pltpu.SemaphoreType.DMA,
        pltpu.SemaphoreType.DMA,
        pltpu.SemaphoreType.DMA,
        pltpu.SemaphoreType.DMA,
    ],
    name="edge_scatter",
)
def _edge_scatter(table, idx, zeros, out, iv, r0, r1, acc, gs0, gs1, ss0, ss1):
    c = lax.axis_index("c")
    s = lax.axis_index("s")
    pltpu.sync_copy(zeros.at[pl.ds(s * RPT, RPT)], acc.at[pl.ds(s * RPT, RPT)])
    plsc.subcore_barrier()

    pltpu.sync_copy(idx.at[c, s, 0], iv)
    g0 = pltpu.async_copy(table.at[iv.at[0, 0]], r0, gs0)
    g1 = pltpu.async_copy(table.at[iv.at[1, 0]], r1, gs1)
    g0.wait()
    pltpu.async_copy(r0, acc.at[iv.at[0, 1]], ss0, add=True)
    g1.wait()
    pltpu.async_copy(r1, acc.at[iv.at[1, 1]], ss1, add=True)

    def body(q, carry):
        pltpu.make_async_copy(r0, acc.at[iv.at[0, 1]], ss0).wait()
        pltpu.make_async_copy(r1, acc.at[iv.at[1, 1]], ss1).wait()
        pltpu.sync_copy(idx.at[c, s, q], iv)
        g0 = pltpu.async_copy(table.at[iv.at[0, 0]], r0, gs0)
        g1 = pltpu.async_copy(table.at[iv.at[1, 0]], r1, gs1)
        g0.wait()
        pltpu.async_copy(r0, acc.at[iv.at[0, 1]], ss0, add=True)
        g1.wait()
        pltpu.async_copy(r1, acc.at[iv.at[1, 1]], ss1, add=True)
        return carry

    lax.fori_loop(1, CH2, body, 0)
    pltpu.make_async_copy(r0, acc.at[iv.at[0, 1]], ss0).wait()
    pltpu.make_async_copy(r1, acc.at[iv.at[1, 1]], ss1).wait()
    plsc.subcore_barrier()
    pltpu.sync_copy(acc.at[pl.ds(s * RPT, RPT)], out.at[c, pl.ds(s * RPT, RPT)])


@functools.partial(
    pl.kernel,
    out_type=jax.ShapeDtypeStruct((NC, NP, WD), jnp.float32),
    mesh=_mesh,
    scratch_types=[
        pltpu.VMEM((CH, K), jnp.int32),
        pltpu.VMEM((K, WD), jnp.float32),
        pltpu.VMEM_SHARED((NP, WD), jnp.float32),
        pltpu.SemaphoreType.DMA,
        pltpu.SemaphoreType.DMA,
    ],
    name="deg_scatter",
)
def _deg_scatter(dst, ones, zeros, out, dst_v, ones_v, acc, ss0, ss1):
    c = lax.axis_index("c")
    s = lax.axis_index("s")
    pltpu.sync_copy(zeros.at[pl.ds(s * RPT, RPT)], acc.at[pl.ds(s * RPT, RPT)])
    pltpu.sync_copy(dst.at[c, s], dst_v)
    pltpu.sync_copy(ones, ones_v)
    plsc.subcore_barrier()

    pltpu.async_copy(ones_v, acc.at[dst_v.at[0]], ss0, add=True)
    pltpu.async_copy(ones_v, acc.at[dst_v.at[1]], ss1, add=True)

    def body(q, carry):
        pltpu.make_async_copy(ones_v, acc.at[dst_v.at[0]], ss0).wait()
        pltpu.async_copy(ones_v, acc.at[dst_v.at[2 * q]], ss0, add=True)
        pltpu.make_async_copy(ones_v, acc.at[dst_v.at[1]], ss1).wait()
        pltpu.async_copy(ones_v, acc.at[dst_v.at[2 * q + 1]], ss1, add=True)
        return carry

    lax.fori_loop(1, CH2, body, 0)
    pltpu.make_async_copy(ones_v, acc.at[dst_v.at[0]], ss0).wait()
    pltpu.make_async_copy(ones_v, acc.at[dst_v.at[1]], ss1).wait()
    plsc.subcore_barrier()
    pltpu.sync_copy(acc.at[pl.ds(s * RPT, RPT)], out.at[c, pl.ds(s * RPT, RPT)])


def _mm_body(x_ref, w_ref, o_ref):
    o_ref[...] = jnp.dot(x_ref[...], w_ref[...],
                         preferred_element_type=jnp.float32)


def _scale_body(deg_ref, xw_ref, dinv_ref, g_ref):
    dinv = lax.rsqrt(deg_ref[:, 0:1] + 1.0)
    dinv_ref[...] = dinv
    g_ref[...] = dinv * xw_ref[...]


def _make_mid_body(fin, fout):
    def mid(agg_ref, g_ref, dinv_ref, b_ref, w_ref, gn_ref):
        dinv = dinv_ref[...]
        h = jnp.maximum(
            dinv * (agg_ref[:, :fin] + g_ref[:, :fin]) + b_ref[...], 0.0)
        gn = dinv * jnp.dot(h, w_ref[...], preferred_element_type=jnp.float32)
        if fout < FW:
            gn = jnp.concatenate(
                [gn, jnp.zeros((N2, FW - fout), jnp.float32)], axis=1)
        gn_ref[...] = gn
    return mid


def _final_body(agg_ref, g_ref, dinv_ref, b_ref, f_ref):
    f_ref[...] = (dinv_ref[...] * (agg_ref[:, :F3] + g_ref[:, :F3])
                  + b_ref[...])


def _tc(body, out_shape, *args):
    return pl.pallas_call(body, out_shape=out_shape)(*args)


def _prep_edges(ei, off):
    src = ei[0].reshape(NS, EPT)
    dst = ei[1].reshape(NS, EPT)
    src = jnp.pad(src, ((0, 0), (0, EPTP - EPT))) + off
    dst = jnp.pad(dst, ((0, 0), (0, EPTP - EPT)), constant_values=N)
    idx = jnp.stack([src.reshape(NS, CH2, 2, K), dst.reshape(NS, CH2, 2, K)],
                    axis=3)
    return idx, dst.reshape(NS, CH, K)


def kernel(features_1, edge_index_1, features_2, edge_index_2, W1, b1, W2, b2,
           W3, b3, att_W, ntn_W, ntn_V, ntn_b, fc_W, fc_b, sc_W, sc_b):
    f32 = jnp.float32
    zrows = jnp.zeros((NP - N, D), f32)
    x_both = jnp.concatenate([features_1, zrows, features_2, zrows], axis=0)

    i1, d1 = _prep_edges(edge_index_1, 0)
    i2, d2 = _prep_edges(edge_index_2, NP)
    idx = jnp.stack([i1, i2])
    dst = jnp.stack([d1, d2])
    zt = jnp.zeros((NP, FW), f32)

    xw = _tc(_mm_body, jax.ShapeDtypeStruct((N2, F1), f32), x_both, W1)
    deg = _deg_scatter(dst, jnp.ones((K, WD), f32), jnp.zeros((NP, WD), f32))
    deg = deg.reshape(N2, WD)

    dinv, g1 = _tc(
        _scale_body,
        (jax.ShapeDtypeStruct((N2, 1), f32), jax.ShapeDtypeStruct((N2, F1), f32)),
        deg, xw)

    a1 = _edge_scatter(g1, idx, zt)
    g2 = _tc(_make_mid_body(F1, F2), jax.ShapeDtypeStruct((N2, FW), f32),
             a1.reshape(N2, FW), g1, dinv, b1[None, :], W2)

    a2 = _edge_scatter(g2, idx, zt)
    g3 = _tc(_make_mid_body(F2, F3), jax.ShapeDtypeStruct((N2, FW), f32),
             a2.reshape(N2, FW), g2, dinv, b2[None, :], W3)

    a3 = _edge_scatter(g3, idx, zt)
    f = _tc(_final_body, jax.ShapeDtypeStruct((N2, F3), f32),
            a3.reshape(N2, FW), g3, dinv, b3[None, :])

    def _pool(h):
        gc = jnp.tanh(jnp.mean(h @ att_W, axis=0))
        sg = jax.nn.sigmoid(h @ gc[:, None])
        return h.T @ sg

    p1 = _pool(f[0:N])
    p2 = _pool(f[NP:NP + N])
    scoring = (p1.T @ ntn_W.reshape(F3, -1)).reshape(F3, -1)
    scoring = scoring.T @ p2
    comb = jnp.concatenate([p1, p2], axis=0)
    block = ntn_V @ comb
    s = jax.nn.relu(scoring + block + ntn_b).T
    s = jax.nn.relu(s @ fc_W + fc_b)
    return jax.nn.sigmoid(s @ sc_W + sc_b)
